# baseline (speedup 1.0000x reference)
import jax
import jax.numpy as jnp
from jax.experimental import pallas as pl

D_MODEL = 1024
D_SPACE = 64
N_F = 256
N_R = 128
N_V = 256
N_USED = N_F + N_R + N_V
TK_F = 8
TK_R = 4
TK_V = 6


def _ce(a, b):
    return jnp.maximum(a, b), jnp.minimum(a, b)


def _clean(c):
    n = len(c)
    d = n // 2
    while d >= 1:
        out = list(c)
        for i in range(n):
            if i % (2 * d) < d:
                out[i], out[i + d] = _ce(c[i], c[i + d])
        c = out
        d //= 2
    return c


def _sort8(r):
    h0, l0 = _ce(r[0], r[1])
    h1, l1 = _ce(r[2], r[3])
    h2, l2 = _ce(r[4], r[5])
    h3, l3 = _ce(r[6], r[7])
    a = _clean([h0, l0, l1, h1])
    b = _clean([h2, l2, l3, h3])
    return _clean(a + b[::-1])


def _merge_keep8(a, b):
    t = [jnp.maximum(a[i], b[7 - i]) for i in range(8)]
    return _clean(t)


def _top8_candidates(lt):
    g = lt.reshape(32, 8, lt.shape[-1])
    rows = [g[i] for i in range(32)]
    runs = [_sort8(rows[8 * j:8 * j + 8]) for j in range(4)]
    t = _merge_keep8(runs[0], runs[1])
    u = _merge_keep8(runs[2], runs[3])
    top = _merge_keep8(t, u)
    return jnp.concatenate(top, axis=0)


def _sort4(r):
    h0, l0 = _ce(r[0], r[1])
    h1, l1 = _ce(r[2], r[3])
    return _clean([h0, l0, l1, h1])


def _top4_candidates(lt):
    g = lt.reshape(16, 8, lt.shape[-1])
    rows = [g[i] for i in range(16)]
    runs = [_sort4(rows[4 * j:4 * j + 4]) for j in range(4)]
    t = _clean([jnp.maximum(runs[0][i], runs[1][3 - i]) for i in range(4)])
    u = _clean([jnp.maximum(runs[2][i], runs[3][3 - i]) for i in range(4)])
    top = _clean([jnp.maximum(t[i], u[3 - i]) for i in range(4)])
    return jnp.concatenate(top, axis=0)


def _thr_from(cand, m, k):
    neg = jnp.float32(-jnp.inf)
    w = jnp.where(cand == m, neg, cand)
    for _ in range(k - 2):
        cm = jnp.max(w, axis=0, keepdims=True)
        w = jnp.where(w == cm, neg, w)
    return jnp.max(w, axis=0, keepdims=True)


def _route_t(lt, k, cand=None):
    if cand is None:
        cand = lt
    m = jnp.max(cand, axis=0, keepdims=True)
    thr = _thr_from(cand, m, k)
    e = jnp.exp(lt - m)
    z = jnp.sum(e, axis=0, keepdims=True)
    kept = jnp.where(lt >= thr, e, 0.0)
    s = jnp.sum(kept, axis=0, keepdims=True)
    out_t = kept * (1.0 / (s + 1e-8 * z))
    return jnp.transpose(out_t)


def _block_kernel(x_ref, w_ref, b_ref, ne_ref, f_ref, r_ref, v_ref):
    x = x_ref[...]
    ht = jax.lax.dot_general(
        w_ref[...], x, (((0,), (1,)), ((), ())), preferred_element_type=jnp.float32
    ) + jnp.transpose(b_ref[...])
    ne = ne_ref[...]
    norm = jnp.sqrt(jnp.sum(ne * ne, axis=-1, keepdims=True))
    ne_n = ne / jnp.maximum(norm, 1e-12)
    lt = jax.lax.dot_general(
        ne_n, ht, (((1,), (0,)), ((), ())), preferred_element_type=jnp.float32
    )
    lf = lt[:N_F]
    lv = lt[N_F + N_R:N_USED]
    f_ref[...] = _route_t(lf, TK_F, _top8_candidates(lf))
    lr = lt[N_F:N_F + N_R]
    r_ref[...] = _route_t(lr, TK_R, _top4_candidates(lr))
    v_ref[...] = _route_t(lv, TK_V, _top8_candidates(lv))


@jax.jit
def kernel(x, importance, W_proj, b_proj, neuron_emb):
    del importance
    B, S, D = x.shape
    T = B * S
    xf = x.reshape(T, D)
    ne = neuron_emb[:N_USED]
    b2 = b_proj.reshape(1, D_SPACE)
    BT = 2048
    f, r, v = pl.pallas_call(
        _block_kernel,
        grid=(T // BT,),
        in_specs=[
            pl.BlockSpec((BT, D_MODEL), lambda i: (i, 0)),
            pl.BlockSpec((D_MODEL, D_SPACE), lambda i: (0, 0)),
            pl.BlockSpec((1, D_SPACE), lambda i: (0, 0)),
            pl.BlockSpec((N_USED, D_SPACE), lambda i: (0, 0)),
        ],
        out_specs=[
            pl.BlockSpec((BT, N_F), lambda i: (i, 0)),
            pl.BlockSpec((BT, N_R), lambda i: (i, 0)),
            pl.BlockSpec((BT, N_V), lambda i: (i, 0)),
        ],
        out_shape=[
            jax.ShapeDtypeStruct((T, N_F), jnp.float32),
            jax.ShapeDtypeStruct((T, N_R), jnp.float32),
            jax.ShapeDtypeStruct((T, N_V), jnp.float32),
        ],
    )(xf, W_proj, b2, ne)
    return (f.reshape(B, S, N_F), r.reshape(B, S, N_R), r.reshape(B, S, N_R), v.reshape(B, S, N_V))


# speedup vs baseline: 1.0378x; 1.0378x over previous
import jax
import jax.numpy as jnp
from jax.experimental import pallas as pl

D_MODEL = 1024
D_SPACE = 64
N_F = 256
N_R = 128
N_V = 256
N_USED = N_F + N_R + N_V
TK_F = 8
TK_R = 4
TK_V = 6


def _ce(a, b):
    return jnp.maximum(a, b), jnp.minimum(a, b)


def _clean(c):
    n = len(c)
    d = n // 2
    while d >= 1:
        out = list(c)
        for i in range(n):
            if i % (2 * d) < d:
                out[i], out[i + d] = _ce(c[i], c[i + d])
        c = out
        d //= 2
    return c


def _sort8(r):
    h0, l0 = _ce(r[0], r[1])
    h1, l1 = _ce(r[2], r[3])
    h2, l2 = _ce(r[4], r[5])
    h3, l3 = _ce(r[6], r[7])
    a = _clean([h0, l0, l1, h1])
    b = _clean([h2, l2, l3, h3])
    return _clean(a + b[::-1])


def _merge_keep8(a, b):
    t = [jnp.maximum(a[i], b[7 - i]) for i in range(8)]
    return _clean(t)


def _top8_candidates(lt):
    g = lt.reshape(32, 8, lt.shape[-1])
    rows = [g[i] for i in range(32)]
    runs = [_sort8(rows[8 * j:8 * j + 8]) for j in range(4)]
    t = _merge_keep8(runs[0], runs[1])
    u = _merge_keep8(runs[2], runs[3])
    top = _merge_keep8(t, u)
    return jnp.concatenate(top, axis=0)


def _sort4(r):
    h0, l0 = _ce(r[0], r[1])
    h1, l1 = _ce(r[2], r[3])
    return _clean([h0, l0, l1, h1])


def _top4_candidates(lt):
    g = lt.reshape(16, 8, lt.shape[-1])
    rows = [g[i] for i in range(16)]
    runs = [_sort4(rows[4 * j:4 * j + 4]) for j in range(4)]
    t = _clean([jnp.maximum(runs[0][i], runs[1][3 - i]) for i in range(4)])
    u = _clean([jnp.maximum(runs[2][i], runs[3][3 - i]) for i in range(4)])
    top = _clean([jnp.maximum(t[i], u[3 - i]) for i in range(4)])
    return jnp.concatenate(top, axis=0)


def _thr_from(cand, m, k):
    neg = jnp.float32(-jnp.inf)
    w = jnp.where(cand == m, neg, cand)
    for _ in range(k - 2):
        cm = jnp.max(w, axis=0, keepdims=True)
        w = jnp.where(w == cm, neg, w)
    return jnp.max(w, axis=0, keepdims=True)


def _route_t(lt, k, cand=None):
    if cand is None:
        cand = lt
    m = jnp.max(cand, axis=0, keepdims=True)
    thr = _thr_from(cand, m, k)
    e = jnp.exp(lt - m)
    z = jnp.sum(e, axis=0, keepdims=True)
    kept = jnp.where(lt >= thr, e, 0.0)
    s = jnp.sum(kept, axis=0, keepdims=True)
    out_t = kept * (1.0 / (s + 1e-8 * z))
    return jnp.transpose(out_t)


def _block_kernel(x_ref, w_ref, b_ref, ne_ref, f_ref, r_ref, v_ref):
    x = x_ref[...]
    ht = jax.lax.dot_general(
        w_ref[...], x, (((0,), (1,)), ((), ())), preferred_element_type=jnp.float32
    ) + jnp.transpose(b_ref[...])
    ne = ne_ref[...]
    norm = jnp.sqrt(jnp.sum(ne * ne, axis=-1, keepdims=True))
    ne_n = ne / jnp.maximum(norm, 1e-12)
    lt = jax.lax.dot_general(
        ne_n, ht, (((1,), (0,)), ((), ())), preferred_element_type=jnp.float32
    )
    lf = lt[:N_F]
    lv = lt[N_F + N_R:N_USED]
    f_ref[...] = _route_t(lf, TK_F, _top8_candidates(lf))
    lr = lt[N_F:N_F + N_R]
    r_ref[...] = _route_t(lr, TK_R, _top4_candidates(lr))
    v_ref[...] = _route_t(lv, TK_V, _top8_candidates(lv))


@jax.jit
def kernel(x, importance, W_proj, b_proj, neuron_emb):
    del importance
    B, S, D = x.shape
    T = B * S
    xf = x.reshape(T, D)
    ne = neuron_emb[:N_USED]
    b2 = b_proj.reshape(1, D_SPACE)
    BT = 1024
    f, r, v = pl.pallas_call(
        _block_kernel,
        grid=(T // BT,),
        in_specs=[
            pl.BlockSpec((BT, D_MODEL), lambda i: (i, 0)),
            pl.BlockSpec((D_MODEL, D_SPACE), lambda i: (0, 0)),
            pl.BlockSpec((1, D_SPACE), lambda i: (0, 0)),
            pl.BlockSpec((N_USED, D_SPACE), lambda i: (0, 0)),
        ],
        out_specs=[
            pl.BlockSpec((BT, N_F), lambda i: (i, 0)),
            pl.BlockSpec((BT, N_R), lambda i: (i, 0)),
            pl.BlockSpec((BT, N_V), lambda i: (i, 0)),
        ],
        out_shape=[
            jax.ShapeDtypeStruct((T, N_F), jnp.float32),
            jax.ShapeDtypeStruct((T, N_R), jnp.float32),
            jax.ShapeDtypeStruct((T, N_V), jnp.float32),
        ],
    )(xf, W_proj, b2, ne)
    return (f.reshape(B, S, N_F), r.reshape(B, S, N_R), r.reshape(B, S, N_R), v.reshape(B, S, N_V))


# speedup vs baseline: 1.0962x; 1.0563x over previous
import jax
import jax.numpy as jnp
from jax.experimental import pallas as pl
from jax.experimental.pallas import tpu as pltpu

D_MODEL = 1024
D_SPACE = 64
N_F = 256
N_R = 128
N_V = 256
N_USED = N_F + N_R + N_V
TK_F = 8
TK_R = 4
TK_V = 6


def _ce(a, b):
    return jnp.maximum(a, b), jnp.minimum(a, b)


def _clean(c):
    n = len(c)
    d = n // 2
    while d >= 1:
        out = list(c)
        for i in range(n):
            if i % (2 * d) < d:
                out[i], out[i + d] = _ce(c[i], c[i + d])
        c = out
        d //= 2
    return c


def _sort8(r):
    h0, l0 = _ce(r[0], r[1])
    h1, l1 = _ce(r[2], r[3])
    h2, l2 = _ce(r[4], r[5])
    h3, l3 = _ce(r[6], r[7])
    a = _clean([h0, l0, l1, h1])
    b = _clean([h2, l2, l3, h3])
    return _clean(a + b[::-1])


def _merge_keep8(a, b):
    t = [jnp.maximum(a[i], b[7 - i]) for i in range(8)]
    return _clean(t)


def _top8_candidates(lt):
    g = lt.reshape(32, 8, lt.shape[-1])
    rows = [g[i] for i in range(32)]
    runs = [_sort8(rows[8 * j:8 * j + 8]) for j in range(4)]
    t = _merge_keep8(runs[0], runs[1])
    u = _merge_keep8(runs[2], runs[3])
    return _merge_keep8(t, u)


def _sort4(r):
    h0, l0 = _ce(r[0], r[1])
    h1, l1 = _ce(r[2], r[3])
    return _clean([h0, l0, l1, h1])


def _top4_candidates(lt):
    g = lt.reshape(16, 8, lt.shape[-1])
    rows = [g[i] for i in range(16)]
    runs = [_sort4(rows[4 * j:4 * j + 4]) for j in range(4)]
    t = _clean([jnp.maximum(runs[0][i], runs[1][3 - i]) for i in range(4)])
    u = _clean([jnp.maximum(runs[2][i], runs[3][3 - i]) for i in range(4)])
    return _clean([jnp.maximum(t[i], u[3 - i]) for i in range(4)])


def _m_thr_from_runs(runs, k):
    neg = jnp.float32(-jnp.inf)
    m = jnp.max(runs[0], axis=0, keepdims=True)
    act = [jnp.where(runs[0] == m, neg, runs[0])]
    for i in range(1, k - 1):
        act.append(runs[i])
        acc = act[0]
        for a in act[1:]:
            acc = jnp.maximum(acc, a)
        cm = jnp.max(acc, axis=0, keepdims=True)
        act = [jnp.where(a == cm, neg, a) for a in act]
    act.append(runs[k - 1])
    acc = act[0]
    for a in act[1:]:
        acc = jnp.maximum(acc, a)
    thr = jnp.max(acc, axis=0, keepdims=True)
    return m, thr


def _route_t(lt, k, runs):
    m, thr = _m_thr_from_runs(runs, k)
    e = jnp.exp(lt - m)
    z = jnp.sum(e, axis=0, keepdims=True)
    kept = jnp.where(lt >= thr, e, 0.0)
    s = jnp.sum(kept, axis=0, keepdims=True)
    out_t = kept * (1.0 / (s + 1e-8 * z))
    return jnp.transpose(out_t)


def _block_kernel(x_ref, w_ref, b_ref, ne_ref, f_ref, r_ref, v_ref, ne_scr):
    @pl.when(pl.program_id(0) == 0)
    def _():
        ne = ne_ref[...]
        norm = jnp.sqrt(jnp.sum(ne * ne, axis=-1, keepdims=True))
        ne_scr[...] = ne / jnp.maximum(norm, 1e-12)

    x = x_ref[...]
    ht = jax.lax.dot_general(
        w_ref[...], x, (((0,), (1,)), ((), ())), preferred_element_type=jnp.float32
    ) + jnp.transpose(b_ref[...])
    ne_n = ne_scr[...]
    lt = jax.lax.dot_general(
        ne_n, ht, (((1,), (0,)), ((), ())), preferred_element_type=jnp.float32
    )
    lf = lt[:N_F]
    lv = lt[N_F + N_R:N_USED]
    f_ref[...] = _route_t(lf, TK_F, _top8_candidates(lf))
    lr = lt[N_F:N_F + N_R]
    r_ref[...] = _route_t(lr, TK_R, _top4_candidates(lr))
    v_ref[...] = _route_t(lv, TK_V, _top8_candidates(lv))


@jax.jit
def kernel(x, importance, W_proj, b_proj, neuron_emb):
    del importance
    B, S, D = x.shape
    T = B * S
    xf = x.reshape(T, D)
    ne = neuron_emb[:N_USED]
    b2 = b_proj.reshape(1, D_SPACE)
    BT = 1024
    f, r, v = pl.pallas_call(
        _block_kernel,
        grid=(T // BT,),
        in_specs=[
            pl.BlockSpec((BT, D_MODEL), lambda i: (i, 0)),
            pl.BlockSpec((D_MODEL, D_SPACE), lambda i: (0, 0)),
            pl.BlockSpec((1, D_SPACE), lambda i: (0, 0)),
            pl.BlockSpec((N_USED, D_SPACE), lambda i: (0, 0)),
        ],
        out_specs=[
            pl.BlockSpec((BT, N_F), lambda i: (i, 0)),
            pl.BlockSpec((BT, N_R), lambda i: (i, 0)),
            pl.BlockSpec((BT, N_V), lambda i: (i, 0)),
        ],
        out_shape=[
            jax.ShapeDtypeStruct((T, N_F), jnp.float32),
            jax.ShapeDtypeStruct((T, N_R), jnp.float32),
            jax.ShapeDtypeStruct((T, N_V), jnp.float32),
        ],
        scratch_shapes=[pltpu.VMEM((N_USED, D_SPACE), jnp.float32)],
    )(xf, W_proj, b2, ne)
    return (f.reshape(B, S, N_F), r.reshape(B, S, N_R), r.reshape(B, S, N_R), v.reshape(B, S, N_V))


# speedup vs baseline: 1.1201x; 1.0218x over previous
import jax
import jax.numpy as jnp
from jax.experimental import pallas as pl
from jax.experimental.pallas import tpu as pltpu

D_MODEL = 1024
D_SPACE = 64
N_F = 256
N_R = 128
N_V = 256
N_USED = N_F + N_R + N_V
TK_F = 8
TK_R = 4
TK_V = 6


def _ce(a, b):
    return jnp.maximum(a, b), jnp.minimum(a, b)


def _clean(c):
    n = len(c)
    d = n // 2
    while d >= 1:
        out = list(c)
        for i in range(n):
            if i % (2 * d) < d:
                out[i], out[i + d] = _ce(c[i], c[i + d])
        c = out
        d //= 2
    return c


def _sort8(r):
    h0, l0 = _ce(r[0], r[1])
    h1, l1 = _ce(r[2], r[3])
    h2, l2 = _ce(r[4], r[5])
    h3, l3 = _ce(r[6], r[7])
    a = _clean([h0, l0, l1, h1])
    b = _clean([h2, l2, l3, h3])
    return _clean(a + b[::-1])


def _merge_keep8(a, b):
    t = [jnp.maximum(a[i], b[7 - i]) for i in range(8)]
    return _clean(t)


def _top8_candidates(lt):
    g = lt.reshape(32, 8, lt.shape[-1])
    rows = [g[i] for i in range(32)]
    runs = [_sort8(rows[8 * j:8 * j + 8]) for j in range(4)]
    t = _merge_keep8(runs[0], runs[1])
    u = _merge_keep8(runs[2], runs[3])
    return _merge_keep8(t, u)


def _sort4(r):
    h0, l0 = _ce(r[0], r[1])
    h1, l1 = _ce(r[2], r[3])
    return _clean([h0, l0, l1, h1])


def _top4_candidates(lt):
    g = lt.reshape(16, 8, lt.shape[-1])
    rows = [g[i] for i in range(16)]
    runs = [_sort4(rows[4 * j:4 * j + 4]) for j in range(4)]
    t = _clean([jnp.maximum(runs[0][i], runs[1][3 - i]) for i in range(4)])
    u = _clean([jnp.maximum(runs[2][i], runs[3][3 - i]) for i in range(4)])
    return _clean([jnp.maximum(t[i], u[3 - i]) for i in range(4)])


def _m_thr_from_runs(runs, k):
    neg = jnp.float32(-jnp.inf)
    m = jnp.max(runs[0], axis=0, keepdims=True)
    act = [jnp.where(runs[0] == m, neg, runs[0])]
    for i in range(1, k - 1):
        act.append(runs[i])
        acc = act[0]
        for a in act[1:]:
            acc = jnp.maximum(acc, a)
        cm = jnp.max(acc, axis=0, keepdims=True)
        act = [jnp.where(a == cm, neg, a) for a in act]
    act.append(runs[k - 1])
    acc = act[0]
    for a in act[1:]:
        acc = jnp.maximum(acc, a)
    thr = jnp.max(acc, axis=0, keepdims=True)
    return m, thr


def _route_t(lt, k, runs):
    m, thr = _m_thr_from_runs(runs, k)
    e = jnp.exp(lt - m)
    kept = jnp.where(lt >= thr, e, 0.0)
    s = jnp.sum(kept, axis=0, keepdims=True)
    out_t = kept * (1.0 / s)
    return jnp.transpose(out_t)


def _block_kernel(x_ref, w_ref, b_ref, ne_ref, f_ref, r_ref, v_ref, ne_scr):
    @pl.when(pl.program_id(0) == 0)
    def _():
        ne = ne_ref[...]
        norm = jnp.sqrt(jnp.sum(ne * ne, axis=-1, keepdims=True))
        ne_scr[...] = ne / jnp.maximum(norm, 1e-12)

    x = x_ref[...]
    ht = jax.lax.dot_general(
        w_ref[...], x, (((0,), (1,)), ((), ())), preferred_element_type=jnp.float32
    ) + jnp.transpose(b_ref[...])
    ne_n = ne_scr[...]
    lt = jax.lax.dot_general(
        ne_n, ht, (((1,), (0,)), ((), ())), preferred_element_type=jnp.float32
    )
    lf = lt[:N_F]
    lv = lt[N_F + N_R:N_USED]
    f_ref[...] = _route_t(lf, TK_F, _top8_candidates(lf))
    lr = lt[N_F:N_F + N_R]
    r_ref[...] = _route_t(lr, TK_R, _top4_candidates(lr))
    v_ref[...] = _route_t(lv, TK_V, _top8_candidates(lv))


@jax.jit
def kernel(x, importance, W_proj, b_proj, neuron_emb):
    del importance
    B, S, D = x.shape
    T = B * S
    xf = x.reshape(T, D)
    ne = neuron_emb[:N_USED]
    b2 = b_proj.reshape(1, D_SPACE)
    BT = 1024
    f, r, v = pl.pallas_call(
        _block_kernel,
        grid=(T // BT,),
        in_specs=[
            pl.BlockSpec((BT, D_MODEL), lambda i: (i, 0)),
            pl.BlockSpec((D_MODEL, D_SPACE), lambda i: (0, 0)),
            pl.BlockSpec((1, D_SPACE), lambda i: (0, 0)),
            pl.BlockSpec((N_USED, D_SPACE), lambda i: (0, 0)),
        ],
        out_specs=[
            pl.BlockSpec((BT, N_F), lambda i: (i, 0)),
            pl.BlockSpec((BT, N_R), lambda i: (i, 0)),
            pl.BlockSpec((BT, N_V), lambda i: (i, 0)),
        ],
        out_shape=[
            jax.ShapeDtypeStruct((T, N_F), jnp.float32),
            jax.ShapeDtypeStruct((T, N_R), jnp.float32),
            jax.ShapeDtypeStruct((T, N_V), jnp.float32),
        ],
        scratch_shapes=[pltpu.VMEM((N_USED, D_SPACE), jnp.float32)],
    )(xf, W_proj, b2, ne)
    return (f.reshape(B, S, N_F), r.reshape(B, S, N_R), r.reshape(B, S, N_R), v.reshape(B, S, N_V))


# speedup vs baseline: 1.1294x; 1.0083x over previous
import jax
import jax.numpy as jnp
from jax.experimental import pallas as pl
from jax.experimental.pallas import tpu as pltpu

D_MODEL = 1024
D_SPACE = 64
N_F = 256
N_R = 128
N_V = 256
N_USED = N_F + N_R + N_V
TK_F = 8
TK_R = 4
TK_V = 6


def _ce(a, b):
    return jnp.maximum(a, b), jnp.minimum(a, b)


def _clean(c):
    n = len(c)
    d = n // 2
    while d >= 1:
        out = list(c)
        for i in range(n):
            if i % (2 * d) < d:
                out[i], out[i + d] = _ce(c[i], c[i + d])
        c = out
        d //= 2
    return c


_NET8 = [(0, 2), (1, 3), (4, 6), (5, 7),
         (0, 4), (1, 5), (2, 6), (3, 7),
         (0, 1), (2, 3), (4, 5), (6, 7),
         (2, 4), (3, 5),
         (1, 4), (3, 6),
         (1, 2), (3, 4), (5, 6)]


def _sort8(r):
    a = list(r)
    for i, j in _NET8:
        a[i], a[j] = _ce(a[i], a[j])
    return a


def _merge_keep8(a, b):
    t = [jnp.maximum(a[i], b[7 - i]) for i in range(8)]
    return _clean(t)


def _top8_candidates(lt):
    g = lt.reshape(32, 8, lt.shape[-1])
    rows = [g[i] for i in range(32)]
    runs = [_sort8(rows[8 * j:8 * j + 8]) for j in range(4)]
    t = _merge_keep8(runs[0], runs[1])
    u = _merge_keep8(runs[2], runs[3])
    return _merge_keep8(t, u)


def _sort4(r):
    a = list(r)
    for i, j in [(0, 1), (2, 3), (0, 2), (1, 3), (1, 2)]:
        a[i], a[j] = _ce(a[i], a[j])
    return a


def _top4_candidates(lt):
    g = lt.reshape(16, 8, lt.shape[-1])
    rows = [g[i] for i in range(16)]
    runs = [_sort4(rows[4 * j:4 * j + 4]) for j in range(4)]
    t = _clean([jnp.maximum(runs[0][i], runs[1][3 - i]) for i in range(4)])
    u = _clean([jnp.maximum(runs[2][i], runs[3][3 - i]) for i in range(4)])
    return _clean([jnp.maximum(t[i], u[3 - i]) for i in range(4)])


def _m_thr_from_runs(runs, k):
    neg = jnp.float32(-jnp.inf)
    m = jnp.max(runs[0], axis=0, keepdims=True)
    act = [jnp.where(runs[0] == m, neg, runs[0])]
    for i in range(1, k - 1):
        act.append(runs[i])
        acc = act[0]
        for a in act[1:]:
            acc = jnp.maximum(acc, a)
        cm = jnp.max(acc, axis=0, keepdims=True)
        act = [jnp.where(a == cm, neg, a) for a in act]
    act.append(runs[k - 1])
    acc = act[0]
    for a in act[1:]:
        acc = jnp.maximum(acc, a)
    thr = jnp.max(acc, axis=0, keepdims=True)
    return m, thr


def _route_t(lt, k, runs):
    m, thr = _m_thr_from_runs(runs, k)
    e = jnp.exp(lt - m)
    kept = jnp.where(lt >= thr, e, 0.0)
    s = jnp.sum(kept, axis=0, keepdims=True)
    out_t = kept * (1.0 / s)
    return jnp.transpose(out_t)


def _block_kernel(x_ref, w_ref, b_ref, ne_ref, f_ref, r_ref, v_ref, ne_scr):
    @pl.when(pl.program_id(0) == 0)
    def _():
        ne = ne_ref[...]
        norm = jnp.sqrt(jnp.sum(ne * ne, axis=-1, keepdims=True))
        ne_scr[...] = ne / jnp.maximum(norm, 1e-12)

    x = x_ref[...]
    ht = jax.lax.dot_general(
        w_ref[...], x, (((0,), (1,)), ((), ())), preferred_element_type=jnp.float32
    ) + jnp.transpose(b_ref[...])
    ne_n = ne_scr[...]
    lt = jax.lax.dot_general(
        ne_n, ht, (((1,), (0,)), ((), ())), preferred_element_type=jnp.float32
    )
    lf = lt[:N_F]
    lv = lt[N_F + N_R:N_USED]
    f_ref[...] = _route_t(lf, TK_F, _top8_candidates(lf))
    lr = lt[N_F:N_F + N_R]
    r_ref[...] = _route_t(lr, TK_R, _top4_candidates(lr))
    v_ref[...] = _route_t(lv, TK_V, _top8_candidates(lv))


@jax.jit
def kernel(x, importance, W_proj, b_proj, neuron_emb):
    del importance
    B, S, D = x.shape
    T = B * S
    xf = x.reshape(T, D)
    ne = neuron_emb[:N_USED]
    b2 = b_proj.reshape(1, D_SPACE)
    BT = 1024
    f, r, v = pl.pallas_call(
        _block_kernel,
        grid=(T // BT,),
        in_specs=[
            pl.BlockSpec((BT, D_MODEL), lambda i: (i, 0)),
            pl.BlockSpec((D_MODEL, D_SPACE), lambda i: (0, 0)),
            pl.BlockSpec((1, D_SPACE), lambda i: (0, 0)),
            pl.BlockSpec((N_USED, D_SPACE), lambda i: (0, 0)),
        ],
        out_specs=[
            pl.BlockSpec((BT, N_F), lambda i: (i, 0)),
            pl.BlockSpec((BT, N_R), lambda i: (i, 0)),
            pl.BlockSpec((BT, N_V), lambda i: (i, 0)),
        ],
        out_shape=[
            jax.ShapeDtypeStruct((T, N_F), jnp.float32),
            jax.ShapeDtypeStruct((T, N_R), jnp.float32),
            jax.ShapeDtypeStruct((T, N_V), jnp.float32),
        ],
        scratch_shapes=[pltpu.VMEM((N_USED, D_SPACE), jnp.float32)],
    )(xf, W_proj, b2, ne)
    return (f.reshape(B, S, N_F), r.reshape(B, S, N_R), r.reshape(B, S, N_R), v.reshape(B, S, N_V))


# speedup vs baseline: 1.1318x; 1.0021x over previous
import jax
import jax.numpy as jnp
from jax.experimental import pallas as pl
from jax.experimental.pallas import tpu as pltpu

D_MODEL = 1024
D_SPACE = 64
N_F = 256
N_R = 128
N_V = 256
N_USED = N_F + N_R + N_V
TK_F = 8
TK_R = 4
TK_V = 6


def _ce(a, b):
    return jnp.maximum(a, b), jnp.minimum(a, b)


def _clean(c):
    n = len(c)
    d = n // 2
    while d >= 1:
        out = list(c)
        for i in range(n):
            if i % (2 * d) < d:
                out[i], out[i + d] = _ce(c[i], c[i + d])
        c = out
        d //= 2
    return c


_NET8 = [(0, 2), (1, 3), (4, 6), (5, 7),
         (0, 4), (1, 5), (2, 6), (3, 7),
         (0, 1), (2, 3), (4, 5), (6, 7),
         (2, 4), (3, 5),
         (1, 4), (3, 6),
         (1, 2), (3, 4), (5, 6)]


def _sort8(r):
    a = list(r)
    for i, j in _NET8:
        a[i], a[j] = _ce(a[i], a[j])
    return a


def _merge_keep8(a, b):
    t = [jnp.maximum(a[i], b[7 - i]) for i in range(8)]
    return _clean(t)


def _top8_candidates(lt):
    g = lt.reshape(32, 8, lt.shape[-1])
    rows = [g[i] for i in range(32)]
    runs = [_sort8(rows[8 * j:8 * j + 8]) for j in range(4)]
    t = _merge_keep8(runs[0], runs[1])
    u = _merge_keep8(runs[2], runs[3])
    return _merge_keep8(t, u)


def _sort4(r):
    a = list(r)
    for i, j in [(0, 1), (2, 3), (0, 2), (1, 3), (1, 2)]:
        a[i], a[j] = _ce(a[i], a[j])
    return a


def _top4_candidates(lt):
    g = lt.reshape(16, 8, lt.shape[-1])
    rows = [g[i] for i in range(16)]
    runs = [_sort4(rows[4 * j:4 * j + 4]) for j in range(4)]
    t = _clean([jnp.maximum(runs[0][i], runs[1][3 - i]) for i in range(4)])
    u = _clean([jnp.maximum(runs[2][i], runs[3][3 - i]) for i in range(4)])
    return _clean([jnp.maximum(t[i], u[3 - i]) for i in range(4)])


def _m_thr_from_runs(runs, k):
    neg = jnp.float32(-jnp.inf)
    m = jnp.max(runs[0], axis=0, keepdims=True)
    act = [jnp.where(runs[0] == m, neg, runs[0])]
    for i in range(1, k - 1):
        act.append(runs[i])
        acc = act[0]
        for a in act[1:]:
            acc = jnp.maximum(acc, a)
        cm = jnp.max(acc, axis=0, keepdims=True)
        act = [jnp.where(a == cm, neg, a) for a in act]
    act.append(runs[k - 1])
    acc = act[0]
    for a in act[1:]:
        acc = jnp.maximum(acc, a)
    thr = jnp.max(acc, axis=0, keepdims=True)
    return m, thr


def _route_t(lt, k, runs):
    m, thr = _m_thr_from_runs(runs, k)
    e = jnp.exp(lt - m)
    kept = jnp.where(lt >= thr, e, 0.0)
    s = jnp.sum(kept, axis=0, keepdims=True)
    out_t = kept * (1.0 / s)
    return jnp.transpose(out_t)


def _block_kernel(x_ref, w_ref, b_ref, ne_ref, f_ref, r_ref, v_ref, ne_scr):
    @pl.when(pl.program_id(0) == 0)
    def _():
        ne = ne_ref[...]
        norm = jnp.sqrt(jnp.sum(ne * ne, axis=-1, keepdims=True))
        ne_scr[...] = ne / jnp.maximum(norm, 1e-12)

    x = x_ref[...]
    ht = jax.lax.dot_general(
        w_ref[...], x, (((0,), (1,)), ((), ())), preferred_element_type=jnp.float32
    ) + jnp.transpose(b_ref[...])
    ne_n = ne_scr[...]
    lt = jax.lax.dot_general(
        ne_n, ht, (((1,), (0,)), ((), ())), preferred_element_type=jnp.float32
    )
    lf = lt[:N_F]
    lv = lt[N_F + N_R:N_USED]
    f_ref[...] = _route_t(lf, TK_F, _top8_candidates(lf))
    lr = lt[N_F:N_F + N_R]
    r_ref[...] = _route_t(lr, TK_R, _top4_candidates(lr))
    v_ref[...] = _route_t(lv, TK_V, _top8_candidates(lv))


@jax.jit
def kernel(x, importance, W_proj, b_proj, neuron_emb):
    del importance
    B, S, D = x.shape
    T = B * S
    xf = x.reshape(T, D)
    ne = neuron_emb[:N_USED]
    b2 = b_proj.reshape(1, D_SPACE)
    BT = 1024
    f, r, v = pl.pallas_call(
        _block_kernel,
        grid=(T // BT,),
        in_specs=[
            pl.BlockSpec((BT, D_MODEL), lambda i: (i, 0)),
            pl.BlockSpec((D_MODEL, D_SPACE), lambda i: (0, 0)),
            pl.BlockSpec((1, D_SPACE), lambda i: (0, 0)),
            pl.BlockSpec((N_USED, D_SPACE), lambda i: (0, 0)),
        ],
        out_specs=[
            pl.BlockSpec((BT, N_F), lambda i: (i, 0)),
            pl.BlockSpec((BT, N_R), lambda i: (i, 0)),
            pl.BlockSpec((BT, N_V), lambda i: (i, 0)),
        ],
        out_shape=[
            jax.ShapeDtypeStruct((T, N_F), jnp.float32),
            jax.ShapeDtypeStruct((T, N_R), jnp.float32),
            jax.ShapeDtypeStruct((T, N_V), jnp.float32),
        ],
        scratch_shapes=[pltpu.VMEM((N_USED, D_SPACE), jnp.float32)],
        compiler_params=pltpu.CompilerParams(
            dimension_semantics=("arbitrary",),
        ),
    )(xf, W_proj, b2, ne)
    return (f.reshape(B, S, N_F), r.reshape(B, S, N_R), r.reshape(B, S, N_R), v.reshape(B, S, N_V))


# speedup vs baseline: 1.1598x; 1.0247x over previous
import jax
import jax.numpy as jnp
from jax.experimental import pallas as pl
from jax.experimental.pallas import tpu as pltpu

D_MODEL = 1024
D_SPACE = 64
N_F = 256
N_R = 128
N_V = 256
N_USED = N_F + N_R + N_V
TK_F = 8
TK_R = 4
TK_V = 6


def _ce(a, b):
    return jnp.maximum(a, b), jnp.minimum(a, b)


def _clean(c):
    n = len(c)
    d = n // 2
    while d >= 1:
        out = list(c)
        for i in range(n):
            if i % (2 * d) < d:
                out[i], out[i + d] = _ce(c[i], c[i + d])
        c = out
        d //= 2
    return c


_NET8 = [(0, 2), (1, 3), (4, 6), (5, 7),
         (0, 4), (1, 5), (2, 6), (3, 7),
         (0, 1), (2, 3), (4, 5), (6, 7),
         (2, 4), (3, 5),
         (1, 4), (3, 6),
         (1, 2), (3, 4), (5, 6)]


def _sort8(r):
    a = list(r)
    for i, j in _NET8:
        a[i], a[j] = _ce(a[i], a[j])
    return a


def _merge_keep8(a, b):
    t = [jnp.maximum(a[i], b[7 - i]) for i in range(8)]
    return _clean(t)


def _top8_candidates(lt):
    g = lt.reshape(32, 8, lt.shape[-1])
    rows = [g[i] for i in range(32)]
    runs = [_sort8(rows[8 * j:8 * j + 8]) for j in range(4)]
    t = _merge_keep8(runs[0], runs[1])
    u = _merge_keep8(runs[2], runs[3])
    return _merge_keep8(t, u)


def _sort4(r):
    a = list(r)
    for i, j in [(0, 1), (2, 3), (0, 2), (1, 3), (1, 2)]:
        a[i], a[j] = _ce(a[i], a[j])
    return a


def _top4_candidates(lt):
    g = lt.reshape(16, 8, lt.shape[-1])
    rows = [g[i] for i in range(16)]
    runs = [_sort4(rows[4 * j:4 * j + 4]) for j in range(4)]
    t = _clean([jnp.maximum(runs[0][i], runs[1][3 - i]) for i in range(4)])
    u = _clean([jnp.maximum(runs[2][i], runs[3][3 - i]) for i in range(4)])
    return _clean([jnp.maximum(t[i], u[3 - i]) for i in range(4)])


def _m_thr_from_runs(runs, k):
    neg = jnp.float32(-jnp.inf)
    m = jnp.max(runs[0], axis=0, keepdims=True)
    act = [jnp.where(runs[0] == m, neg, runs[0])]
    for i in range(1, k - 1):
        act.append(runs[i])
        acc = act[0]
        for a in act[1:]:
            acc = jnp.maximum(acc, a)
        cm = jnp.max(acc, axis=0, keepdims=True)
        act = [jnp.where(a == cm, neg, a) for a in act]
    act.append(runs[k - 1])
    acc = act[0]
    for a in act[1:]:
        acc = jnp.maximum(acc, a)
    thr = jnp.max(acc, axis=0, keepdims=True)
    return m, thr


def _route_t(lt, k, runs):
    m, thr = _m_thr_from_runs(runs, k)
    s = jnp.zeros_like(m)
    for j in range(k):
        s = s + jnp.where(runs[j] >= thr, jnp.exp(runs[j] - m), 0.0)
    s = jnp.sum(s, axis=0, keepdims=True)
    off = m + jnp.log(s)
    out_t = jnp.where(lt >= thr, jnp.exp(lt - off), 0.0)
    return jnp.transpose(out_t)


def _block_kernel(x_ref, w_ref, b_ref, ne_ref, f_ref, r_ref, v_ref, ne_scr):
    @pl.when(pl.program_id(0) == 0)
    def _():
        ne = ne_ref[...]
        norm = jnp.sqrt(jnp.sum(ne * ne, axis=-1, keepdims=True))
        ne_scr[...] = ne / jnp.maximum(norm, 1e-12)

    x = x_ref[...]
    ht = jax.lax.dot_general(
        w_ref[...], x, (((0,), (1,)), ((), ())), preferred_element_type=jnp.float32
    ) + jnp.transpose(b_ref[...])
    ne_n = ne_scr[...]
    lt = jax.lax.dot_general(
        ne_n, ht, (((1,), (0,)), ((), ())), preferred_element_type=jnp.float32
    )
    lf = lt[:N_F]
    lv = lt[N_F + N_R:N_USED]
    f_ref[...] = _route_t(lf, TK_F, _top8_candidates(lf))
    lr = lt[N_F:N_F + N_R]
    r_ref[...] = _route_t(lr, TK_R, _top4_candidates(lr))
    v_ref[...] = _route_t(lv, TK_V, _top8_candidates(lv))


@jax.jit
def kernel(x, importance, W_proj, b_proj, neuron_emb):
    del importance
    B, S, D = x.shape
    T = B * S
    xf = x.reshape(T, D)
    ne = neuron_emb[:N_USED]
    b2 = b_proj.reshape(1, D_SPACE)
    BT = 1024
    f, r, v = pl.pallas_call(
        _block_kernel,
        grid=(T // BT,),
        in_specs=[
            pl.BlockSpec((BT, D_MODEL), lambda i: (i, 0)),
            pl.BlockSpec((D_MODEL, D_SPACE), lambda i: (0, 0)),
            pl.BlockSpec((1, D_SPACE), lambda i: (0, 0)),
            pl.BlockSpec((N_USED, D_SPACE), lambda i: (0, 0)),
        ],
        out_specs=[
            pl.BlockSpec((BT, N_F), lambda i: (i, 0)),
            pl.BlockSpec((BT, N_R), lambda i: (i, 0)),
            pl.BlockSpec((BT, N_V), lambda i: (i, 0)),
        ],
        out_shape=[
            jax.ShapeDtypeStruct((T, N_F), jnp.float32),
            jax.ShapeDtypeStruct((T, N_R), jnp.float32),
            jax.ShapeDtypeStruct((T, N_V), jnp.float32),
        ],
        scratch_shapes=[pltpu.VMEM((N_USED, D_SPACE), jnp.float32)],
        compiler_params=pltpu.CompilerParams(
            dimension_semantics=("arbitrary",),
        ),
    )(xf, W_proj, b2, ne)
    return (f.reshape(B, S, N_F), r.reshape(B, S, N_R), r.reshape(B, S, N_R), v.reshape(B, S, N_V))
